# baseline (speedup 1.0000x reference)
import functools

import jax
import jax.numpy as jnp
from jax import lax
from jax.experimental import pallas as pl
from jax.experimental.pallas import tpu as pltpu
from jax.experimental.pallas import tpu_sc as plsc

N = 10000
E = 320000
B = 64
IN_DIM = 128
H = 146
HP = 160
HH = 80
EPS = 1e-5

NC = 2
NS = 16
KR = 125
K = 128
CHUNKS = E // (NS * KR)
PC = 20
RING = 4
NP = 10112
STRIPE = NP // NS

_PREC = jax.lax.Precision.HIGHEST


G = 10
RB = N // G


def _embed_body(x_ref, w_ref, b_ref, o_ref):
    e = (jnp.dot(x_ref[...], w_ref[...], preferred_element_type=jnp.float32,
                 precision=_PREC)
         + b_ref[...])
    o_ref[0] = e[:, :HH]
    o_ref[1] = e[:, HH:]


def _mm_stats_body(a0_ref, a1_ref, w_ref, t_ref, s_ref, q_ref):
    t = (jnp.dot(a0_ref[0], w_ref[0:HH, :],
                 preferred_element_type=jnp.float32, precision=_PREC)
         + jnp.dot(a1_ref[0], w_ref[HH:HP, :],
                   preferred_element_type=jnp.float32, precision=_PREC))
    t_ref[...] = t
    s_ref[...] = jnp.sum(t, axis=0, keepdims=True)[None]
    q_ref[...] = jnp.sum(t * t, axis=0, keepdims=True)[None]


def _norm_body(t_ref, s_ref, q_ref, g_ref, be_ref, o_ref):
    mean = jnp.sum(s_ref[...], axis=0) / N
    msq = jnp.sum(q_ref[...], axis=0) / N
    var = msq - mean * mean
    h = g_ref[...] * (t_ref[...] - mean) * jax.lax.rsqrt(var + EPS) + be_ref[...]
    h = jnp.maximum(h, 0.0)
    o_ref[0] = h[:, :HH]
    o_ref[1] = h[:, HH:]


def _pool_body(t_ref, s_ref, q_ref, g_ref, be_ref, batch_ref, o_ref,
               sums_acc, cnt_acc):
    i = pl.program_id(0)
    mean = jnp.sum(s_ref[...], axis=0) / N
    msq = jnp.sum(q_ref[...], axis=0) / N
    var = msq - mean * mean
    h = g_ref[...] * (t_ref[...] - mean) * jax.lax.rsqrt(var + EPS) + be_ref[...]
    h = jnp.maximum(h, 0.0)
    row = batch_ref[0, 0:1, :]
    ids = jax.lax.broadcasted_iota(jnp.int32, (B, RB), 0)
    m = (ids == jnp.broadcast_to(row, (B, RB))).astype(jnp.float32)
    ps = jnp.dot(m, h, preferred_element_type=jnp.float32, precision=_PREC)
    pc = jnp.sum(m, axis=1, keepdims=True)

    @pl.when(i == 0)
    def _init():
        sums_acc[...] = ps
        cnt_acc[...] = pc

    @pl.when(i > 0)
    def _acc():
        sums_acc[...] += ps
        cnt_acc[...] += pc

    @pl.when(i == G - 1)
    def _fin():
        o_ref[...] = sums_acc[...] / jnp.maximum(cnt_acc[...], 1.0)


def _tc_embed(x, w_pad, b_pad):
    return pl.pallas_call(
        _embed_body,
        out_shape=jax.ShapeDtypeStruct((NC, N, HH), jnp.float32),
    )(x, w_pad, b_pad)


def _tc_mm_stats(agg, w_pad):
    return pl.pallas_call(
        _mm_stats_body,
        grid=(G,),
        in_specs=[
            pl.BlockSpec((1, RB, HH), lambda i: (0, i, 0)),
            pl.BlockSpec((1, RB, HH), lambda i: (1, i, 0)),
            pl.BlockSpec((HP, HP), lambda i: (0, 0)),
        ],
        out_specs=[
            pl.BlockSpec((RB, HP), lambda i: (i, 0)),
            pl.BlockSpec((1, 1, HP), lambda i: (i, 0, 0)),
            pl.BlockSpec((1, 1, HP), lambda i: (i, 0, 0)),
        ],
        out_shape=[
            jax.ShapeDtypeStruct((N, HP), jnp.float32),
            jax.ShapeDtypeStruct((G, 1, HP), jnp.float32),
            jax.ShapeDtypeStruct((G, 1, HP), jnp.float32),
        ],
    )(agg, agg, w_pad)


def _tc_norm(t, s, q, g_pad, be_pad):
    return pl.pallas_call(
        _norm_body,
        grid=(G,),
        in_specs=[
            pl.BlockSpec((RB, HP), lambda i: (i, 0)),
            pl.BlockSpec((G, 1, HP), lambda i: (0, 0, 0)),
            pl.BlockSpec((G, 1, HP), lambda i: (0, 0, 0)),
            pl.BlockSpec((1, HP), lambda i: (0, 0)),
            pl.BlockSpec((1, HP), lambda i: (0, 0)),
        ],
        out_specs=pl.BlockSpec((NC, RB, HH), lambda i: (0, i, 0)),
        out_shape=jax.ShapeDtypeStruct((NC, N, HH), jnp.float32),
    )(t, s, q, g_pad, be_pad)


def _tc_pool(t, s, q, g_pad, be_pad, batch8):
    return pl.pallas_call(
        _pool_body,
        grid=(G,),
        in_specs=[
            pl.BlockSpec((RB, HP), lambda i: (i, 0)),
            pl.BlockSpec((G, 1, HP), lambda i: (0, 0, 0)),
            pl.BlockSpec((G, 1, HP), lambda i: (0, 0, 0)),
            pl.BlockSpec((1, HP), lambda i: (0, 0)),
            pl.BlockSpec((1, HP), lambda i: (0, 0)),
            pl.BlockSpec((1, 8, RB), lambda i: (i, 0, 0)),
        ],
        out_specs=pl.BlockSpec((B, HP), lambda i: (0, 0)),
        out_shape=jax.ShapeDtypeStruct((B, HP), jnp.float32),
        scratch_shapes=[
            pltpu.VMEM((B, HP), jnp.float32),
            pltpu.VMEM((B, 1), jnp.float32),
        ],
    )(t, s, q, g_pad, be_pad, batch8)


def _sc_agg_body(h_hbm, src_hbm, dst_hbm, z_hbm, out_hbm,
                 src_v, dst_v, *rest):
    rows = rest[0:RING]
    agg_sh = rest[RING]
    gsem = rest[RING + 1:2 * RING + 1]
    ssem = rest[2 * RING + 1:3 * RING + 1]
    c = lax.axis_index("c")
    s = lax.axis_index("s")
    base = s * STRIPE
    rows0 = rows[0]

    h_c = h_hbm.at[c]

    def sidx(i):
        return src_v.at[i, 0]

    def didx(i):
        return dst_v.at[i, 0]

    pltpu.sync_copy(z_hbm, rows0)
    for j in range(4):
        pltpu.sync_copy(rows0, agg_sh.at[pl.ds(base + j * K, K)])
    pltpu.sync_copy(rows0.at[pl.ds(0, 120)],
                    agg_sh.at[pl.ds(base + 4 * K, 120)])
    plsc.subcore_barrier()

    def gissue(i, b):
        pltpu.async_copy(h_c.at[sidx(i)], rows[b], gsem[b])

    def gwait(i, b):
        pltpu.make_async_copy(h_c.at[sidx(i)], rows[b], gsem[b]).wait()

    def sissue(i, b):
        pltpu.async_copy(rows[b], agg_sh.at[didx(i)], ssem[b], add=True)

    def swait(i, b):
        pltpu.make_async_copy(rows[b], agg_sh.at[didx(i)], ssem[b]).wait()

    for phase in range(CHUNKS // PC):
        pltpu.sync_copy(src_hbm.at[s, pl.ds(phase * PC, PC)], src_v)
        pltpu.sync_copy(dst_hbm.at[s, pl.ds(phase * PC, PC)], dst_v)

        gissue(0, 0)
        gissue(1, 1)
        gwait(0, 0); sissue(0, 0); gissue(2, 2)
        gwait(1, 1); sissue(1, 1); gissue(3, 3)
        gwait(2, 2); sissue(2, 2); swait(0, 0); gissue(4, 0)
        gwait(3, 3); sissue(3, 3); swait(1, 1); gissue(5, 1)

        def round_(g, carry):
            base = RING * g
            for k in range(RING):
                i = base + k
                gwait(i, k)
                sissue(i, k)
                b2 = (k + 2) % RING
                swait(i - 2, b2)
                gissue(i + 2, b2)
            return carry

        lax.fori_loop(1, PC // RING - 1, round_, 0)
        gwait(PC - 4, 0); sissue(PC - 4, 0); swait(PC - 6, 2); gissue(PC - 2, 2)
        gwait(PC - 3, 1); sissue(PC - 3, 1); swait(PC - 5, 3); gissue(PC - 1, 3)
        gwait(PC - 2, 2); sissue(PC - 2, 2); swait(PC - 4, 0)
        gwait(PC - 1, 3); sissue(PC - 1, 3); swait(PC - 3, 1)
        swait(PC - 2, 2)
        swait(PC - 1, 3)
    plsc.subcore_barrier()

    for j in range(4):
        pltpu.sync_copy(agg_sh.at[pl.ds(base + j * K, K)],
                        out_hbm.at[c, pl.ds(base + j * K, K)])
    pltpu.sync_copy(agg_sh.at[pl.ds(base + 4 * K, 120)],
                    out_hbm.at[c, pl.ds(base + 4 * K, 120)])


@functools.cache
def _make_sc_agg():
    return pl.kernel(
        _sc_agg_body,
        mesh=plsc.VectorSubcoreMesh(core_axis_name="c", subcore_axis_name="s"),
        compiler_params=pltpu.CompilerParams(use_tc_tiling_on_sc=False),
        out_type=jax.ShapeDtypeStruct((NC, NP, HH), jnp.float32),
        scratch_types=[
            pltpu.VMEM((PC, 1, K), jnp.int32),
            pltpu.VMEM((PC, 1, K), jnp.int32),
        ] + [pltpu.VMEM((K, HH), jnp.float32) for _ in range(RING)]
        + [pltpu.VMEM_SHARED((NP, HH), jnp.float32)]
        + [pltpu.SemaphoreType.DMA] * (2 * RING),
    )


def kernel(x, edge_index, batch, W_emb, b_emb, W1, b1, W2, b2, W3, b3, W4, b4,
           g1, be1, g2, be2, g3, be3, g4, be4):
    del b1, b2, b3, b4

    npad = K - KR
    src = jnp.pad(edge_index[0].reshape(NS, CHUNKS, KR),
                  ((0, 0), (0, 0), (0, npad)))
    junk = (N + (jnp.arange(CHUNKS)[:, None] * npad
                 + jnp.arange(npad)[None, :]) % (NP - N)).astype(jnp.int32)
    junk = jnp.broadcast_to(junk[None], (NS, CHUNKS, npad))
    dst = jnp.concatenate([edge_index[1].reshape(NS, CHUNKS, KR), junk],
                          axis=-1)
    src = src.reshape(NS, CHUNKS, 1, K)
    dst = dst.reshape(NS, CHUNKS, 1, K)
    zrows = jnp.zeros((K, HH), jnp.float32)
    batch8 = jnp.broadcast_to(batch.reshape(G, 1, RB), (G, 8, RB))

    pad_c = HP - H
    w_emb_p = jnp.pad(W_emb, ((0, 0), (0, pad_c)))
    b_emb_p = jnp.pad(b_emb, (0, pad_c)).reshape(1, HP)

    def pad_layer(w, g, be):
        return (jnp.pad(w, ((0, pad_c), (0, pad_c))),
                jnp.pad(g, (0, pad_c)).reshape(1, HP),
                jnp.pad(be, (0, pad_c)).reshape(1, HP))

    layers = [pad_layer(W1, g1, be1), pad_layer(W2, g2, be2),
              pad_layer(W3, g3, be3), pad_layer(W4, g4, be4)]

    h = _tc_embed(x, w_emb_p, b_emb_p)
    for li, (w_p, g_p, be_p) in enumerate(layers):
        agg = _make_sc_agg()(h, src, dst, zrows)
        t, s, q = _tc_mm_stats(agg, w_p)
        if li < 3:
            h = _tc_norm(t, s, q, g_p, be_p)
        else:
            hg = _tc_pool(t, s, q, g_p, be_p, batch8)
    return hg[:, :H]


# speedup vs baseline: 1.3199x; 1.3199x over previous
import functools

import jax
import jax.numpy as jnp
from jax import lax
from jax.experimental import pallas as pl
from jax.experimental.pallas import tpu as pltpu
from jax.experimental.pallas import tpu_sc as plsc

N = 10000
E = 320000
B = 64
IN_DIM = 128
H = 146
HP = 160
HH = 80
EPS = 1e-5

NC = 2
NS = 16
KR = 125
K = 128
CHUNKS = E // (NS * KR)
PC = 20
RING = 2
NP = 10112
STRIPE = NP // NS

_PREC = jax.lax.Precision.HIGHEST


G = 10
RB = N // G


def _embed_body(x_ref, w_ref, b_ref, o_ref):
    e = (jnp.dot(x_ref[...], w_ref[...], preferred_element_type=jnp.float32,
                 precision=_PREC)
         + b_ref[...])
    o_ref[0] = e[:, :HH]
    o_ref[1] = e[:, HH:]


def _mm_stats_body(a0_ref, a1_ref, w_ref, t_ref, s_ref, q_ref):
    t = (jnp.dot(a0_ref[0], w_ref[0:HH, :],
                 preferred_element_type=jnp.float32, precision=_PREC)
         + jnp.dot(a1_ref[0], w_ref[HH:HP, :],
                   preferred_element_type=jnp.float32, precision=_PREC))
    t_ref[...] = t
    s_ref[...] = jnp.sum(t, axis=0, keepdims=True)[None]
    q_ref[...] = jnp.sum(t * t, axis=0, keepdims=True)[None]


def _norm_body(t_ref, s_ref, q_ref, g_ref, be_ref, o_ref):
    mean = jnp.sum(s_ref[...], axis=0) / N
    msq = jnp.sum(q_ref[...], axis=0) / N
    var = msq - mean * mean
    h = g_ref[...] * (t_ref[...] - mean) * jax.lax.rsqrt(var + EPS) + be_ref[...]
    h = jnp.maximum(h, 0.0)
    o_ref[0] = h[:, :HH]
    o_ref[1] = h[:, HH:]


def _pool_body(t_ref, s_ref, q_ref, g_ref, be_ref, batch_ref, o_ref,
               sums_acc, cnt_acc):
    i = pl.program_id(0)
    mean = jnp.sum(s_ref[...], axis=0) / N
    msq = jnp.sum(q_ref[...], axis=0) / N
    var = msq - mean * mean
    h = g_ref[...] * (t_ref[...] - mean) * jax.lax.rsqrt(var + EPS) + be_ref[...]
    h = jnp.maximum(h, 0.0)
    row = batch_ref[0, 0:1, :]
    ids = jax.lax.broadcasted_iota(jnp.int32, (B, RB), 0)
    m = (ids == jnp.broadcast_to(row, (B, RB))).astype(jnp.float32)
    ps = jnp.dot(m, h, preferred_element_type=jnp.float32, precision=_PREC)
    pc = jnp.sum(m, axis=1, keepdims=True)

    @pl.when(i == 0)
    def _init():
        sums_acc[...] = ps
        cnt_acc[...] = pc

    @pl.when(i > 0)
    def _acc():
        sums_acc[...] += ps
        cnt_acc[...] += pc

    @pl.when(i == G - 1)
    def _fin():
        o_ref[...] = sums_acc[...] / jnp.maximum(cnt_acc[...], 1.0)


def _tc_embed(x, w_pad, b_pad):
    return pl.pallas_call(
        _embed_body,
        out_shape=jax.ShapeDtypeStruct((NC, N, HH), jnp.float32),
    )(x, w_pad, b_pad)


def _tc_mm_stats(agg, w_pad):
    return pl.pallas_call(
        _mm_stats_body,
        grid=(G,),
        in_specs=[
            pl.BlockSpec((1, RB, HH), lambda i: (0, i, 0)),
            pl.BlockSpec((1, RB, HH), lambda i: (1, i, 0)),
            pl.BlockSpec((HP, HP), lambda i: (0, 0)),
        ],
        out_specs=[
            pl.BlockSpec((RB, HP), lambda i: (i, 0)),
            pl.BlockSpec((1, 1, HP), lambda i: (i, 0, 0)),
            pl.BlockSpec((1, 1, HP), lambda i: (i, 0, 0)),
        ],
        out_shape=[
            jax.ShapeDtypeStruct((N, HP), jnp.float32),
            jax.ShapeDtypeStruct((G, 1, HP), jnp.float32),
            jax.ShapeDtypeStruct((G, 1, HP), jnp.float32),
        ],
    )(agg, agg, w_pad)


def _tc_norm(t, s, q, g_pad, be_pad):
    return pl.pallas_call(
        _norm_body,
        grid=(G,),
        in_specs=[
            pl.BlockSpec((RB, HP), lambda i: (i, 0)),
            pl.BlockSpec((G, 1, HP), lambda i: (0, 0, 0)),
            pl.BlockSpec((G, 1, HP), lambda i: (0, 0, 0)),
            pl.BlockSpec((1, HP), lambda i: (0, 0)),
            pl.BlockSpec((1, HP), lambda i: (0, 0)),
        ],
        out_specs=pl.BlockSpec((NC, RB, HH), lambda i: (0, i, 0)),
        out_shape=jax.ShapeDtypeStruct((NC, N, HH), jnp.float32),
    )(t, s, q, g_pad, be_pad)


def _tc_pool(t, s, q, g_pad, be_pad, batch8):
    return pl.pallas_call(
        _pool_body,
        grid=(G,),
        in_specs=[
            pl.BlockSpec((RB, HP), lambda i: (i, 0)),
            pl.BlockSpec((G, 1, HP), lambda i: (0, 0, 0)),
            pl.BlockSpec((G, 1, HP), lambda i: (0, 0, 0)),
            pl.BlockSpec((1, HP), lambda i: (0, 0)),
            pl.BlockSpec((1, HP), lambda i: (0, 0)),
            pl.BlockSpec((1, 8, RB), lambda i: (i, 0, 0)),
        ],
        out_specs=pl.BlockSpec((B, HP), lambda i: (0, 0)),
        out_shape=jax.ShapeDtypeStruct((B, HP), jnp.float32),
        scratch_shapes=[
            pltpu.VMEM((B, HP), jnp.float32),
            pltpu.VMEM((B, 1), jnp.float32),
        ],
    )(t, s, q, g_pad, be_pad, batch8)


def _sc_agg_body(h_hbm, src_hbm, dst_hbm, z_hbm, out_hbm,
                 src_v, dst_v, *rest):
    rows = rest[0:RING]
    h_sp = rest[RING]
    agg_sh = rest[RING + 1]
    gsem = rest[RING + 2:2 * RING + 2]
    ssem = rest[2 * RING + 2:3 * RING + 2]
    c = lax.axis_index("c")
    s = lax.axis_index("s")
    base = s * STRIPE
    rows0 = rows[0]

    h_c = h_hbm.at[c]

    @pl.when(s < NS - 1)
    def _stage():
        pltpu.sync_copy(h_c.at[pl.ds(s * STRIPE, STRIPE)],
                        h_sp.at[pl.ds(s * STRIPE, STRIPE)])

    @pl.when(s == NS - 1)
    def _stage_last():
        r = N - (NS - 1) * STRIPE
        pltpu.sync_copy(h_c.at[pl.ds((NS - 1) * STRIPE, r)],
                        h_sp.at[pl.ds((NS - 1) * STRIPE, r)])

    def sidx(i):
        return src_v.at[i, 0]

    def didx(i):
        return dst_v.at[i, 0]

    pltpu.sync_copy(z_hbm, rows0)
    for j in range(4):
        pltpu.sync_copy(rows0, agg_sh.at[pl.ds(base + j * K, K)])
    pltpu.sync_copy(rows0.at[pl.ds(0, 120)],
                    agg_sh.at[pl.ds(base + 4 * K, 120)])
    plsc.subcore_barrier()

    for phase in range(CHUNKS // PC):
        pltpu.sync_copy(src_hbm.at[s, pl.ds(phase * PC, PC)], src_v)
        pltpu.sync_copy(dst_hbm.at[s, pl.ds(phase * PC, PC)], dst_v)

        for b in range(RING):
            pltpu.async_copy(h_sp.at[sidx(b)], rows[b], gsem[b])

        def round_(g, carry):
            for b in range(RING):
                i = RING * g + b
                pltpu.make_async_copy(h_sp.at[sidx(i)], rows[b],
                                      gsem[b]).wait()
                pltpu.async_copy(rows[b], agg_sh.at[didx(i)], ssem[b],
                                 add=True).wait()
                pltpu.async_copy(h_sp.at[sidx(i + RING)], rows[b], gsem[b])
            return carry

        lax.fori_loop(0, PC // RING - 1, round_, 0)
        for b in range(RING):
            i = PC - RING + b
            pltpu.make_async_copy(h_sp.at[sidx(i)], rows[b], gsem[b]).wait()
            pltpu.async_copy(rows[b], agg_sh.at[didx(i)], ssem[b],
                             add=True).wait()
    plsc.subcore_barrier()

    for j in range(4):
        pltpu.sync_copy(agg_sh.at[pl.ds(base + j * K, K)],
                        out_hbm.at[c, pl.ds(base + j * K, K)])
    pltpu.sync_copy(agg_sh.at[pl.ds(base + 4 * K, 120)],
                    out_hbm.at[c, pl.ds(base + 4 * K, 120)])


@functools.cache
def _make_sc_agg():
    return pl.kernel(
        _sc_agg_body,
        mesh=plsc.VectorSubcoreMesh(core_axis_name="c", subcore_axis_name="s"),
        compiler_params=pltpu.CompilerParams(use_tc_tiling_on_sc=False),
        out_type=jax.ShapeDtypeStruct((NC, NP, HH), jnp.float32),
        scratch_types=[
            pltpu.VMEM((PC, 1, K), jnp.int32),
            pltpu.VMEM((PC, 1, K), jnp.int32),
        ] + [pltpu.VMEM((K, HH), jnp.float32) for _ in range(RING)]
        + [pltpu.VMEM_SHARED((N, HH), jnp.float32)]
        + [pltpu.VMEM_SHARED((NP, HH), jnp.float32)]
        + [pltpu.SemaphoreType.DMA] * (2 * RING),
    )


def kernel(x, edge_index, batch, W_emb, b_emb, W1, b1, W2, b2, W3, b3, W4, b4,
           g1, be1, g2, be2, g3, be3, g4, be4):
    del b1, b2, b3, b4

    npad = K - KR
    src = jnp.pad(edge_index[0].reshape(NS, CHUNKS, KR),
                  ((0, 0), (0, 0), (0, npad)))
    junk = (N + (jnp.arange(CHUNKS)[:, None] * npad
                 + jnp.arange(npad)[None, :]) % (NP - N)).astype(jnp.int32)
    junk = jnp.broadcast_to(junk[None], (NS, CHUNKS, npad))
    dst = jnp.concatenate([edge_index[1].reshape(NS, CHUNKS, KR), junk],
                          axis=-1)
    src = src.reshape(NS, CHUNKS, 1, K)
    dst = dst.reshape(NS, CHUNKS, 1, K)
    zrows = jnp.zeros((K, HH), jnp.float32)
    batch8 = jnp.broadcast_to(batch.reshape(G, 1, RB), (G, 8, RB))

    pad_c = HP - H
    w_emb_p = jnp.pad(W_emb, ((0, 0), (0, pad_c)))
    b_emb_p = jnp.pad(b_emb, (0, pad_c)).reshape(1, HP)

    def pad_layer(w, g, be):
        return (jnp.pad(w, ((0, pad_c), (0, pad_c))),
                jnp.pad(g, (0, pad_c)).reshape(1, HP),
                jnp.pad(be, (0, pad_c)).reshape(1, HP))

    layers = [pad_layer(W1, g1, be1), pad_layer(W2, g2, be2),
              pad_layer(W3, g3, be3), pad_layer(W4, g4, be4)]

    h = _tc_embed(x, w_emb_p, b_emb_p)
    for li, (w_p, g_p, be_p) in enumerate(layers):
        agg = _make_sc_agg()(h, src, dst, zrows)
        t, s, q = _tc_mm_stats(agg, w_p)
        if li < 3:
            h = _tc_norm(t, s, q, g_p, be_p)
        else:
            hg = _tc_pool(t, s, q, g_p, be_p, batch8)
    return hg[:, :H]
